# Initial kernel scaffold; baseline (speedup 1.0000x reference)
import jax
import jax.numpy as jnp
from jax.experimental import pallas as pl


def kernel(x, edge_index_rel0, edge_index_rel1, edge_index_rel2, W_rel0, W_rel1, W_rel2):
    raise NotImplementedError("write your pallas kernel here")


# speedup vs baseline: 3.0445x; 3.0445x over previous
import functools

import jax
import jax.numpy as jnp
from jax import lax
from jax.experimental import pallas as pl
from jax.experimental.pallas import tpu as pltpu
from jax.experimental.pallas import tpu_sc as plsc

N_NODES = 10000
D_FEAT = 128
N_EDGES = 160000
N_REL = 3

NC, NS = 2, 16
CHUNK = 128
ROWS = 1280
ROWS_PER_TILE = ROWS // (NC * NS)
PAD_E = ROWS * CHUNK - N_EDGES
ACC_ROWS = 10240
SHARE = ACC_ROWS // NS

_mesh = plsc.VectorSubcoreMesh(core_axis_name="c", subcore_axis_name="s")


@functools.partial(
    pl.kernel,
    out_type=(
        jax.ShapeDtypeStruct((N_REL, NC, ACC_ROWS, D_FEAT), jnp.float32),
        jax.ShapeDtypeStruct((N_REL, NC, ACC_ROWS), jnp.float32),
    ),
    mesh=_mesh,
    scratch_types=[
        pltpu.VMEM((ROWS_PER_TILE, CHUNK), jnp.int32),
        pltpu.VMEM((ROWS_PER_TILE, CHUNK), jnp.int32),
        pltpu.VMEM((CHUNK, D_FEAT), jnp.float32),
        pltpu.VMEM((CHUNK,), jnp.float32),
        pltpu.VMEM((64, D_FEAT), jnp.float32),
        pltpu.VMEM((SHARE,), jnp.float32),
        pltpu.VMEM_SHARED((ACC_ROWS, D_FEAT), jnp.float32),
        pltpu.VMEM_SHARED((ACC_ROWS,), jnp.float32),
    ],
)
def _sc_aggregate(x_hbm, src_hbm, dst_hbm, ones_hbm, zacc_hbm, zdeg_hbm,
                  acc_out, deg_out,
                  src_v, dst_v, rows_v, ones_v, zacc_v, zdeg_v,
                  acc_sh, deg_sh):
    c = lax.axis_index("c")
    s = lax.axis_index("s")
    wid = s * NC + c

    pltpu.sync_copy(ones_hbm, ones_v)
    pltpu.sync_copy(zacc_hbm, zacc_v)
    pltpu.sync_copy(zdeg_hbm, zdeg_v)

    for r in range(N_REL):
        z0 = s * SHARE

        @pl.loop(0, SHARE // 64)
        def _(k):
            pltpu.sync_copy(zacc_v, acc_sh.at[pl.ds(z0 + k * 64, 64)])

        pltpu.sync_copy(zdeg_v, deg_sh.at[pl.ds(z0, SHARE)])
        row0 = wid * ROWS_PER_TILE
        pltpu.sync_copy(src_hbm.at[r, pl.ds(row0, ROWS_PER_TILE)], src_v)
        pltpu.sync_copy(dst_hbm.at[r, pl.ds(row0, ROWS_PER_TILE)], dst_v)
        plsc.subcore_barrier()

        @pl.loop(0, ROWS_PER_TILE)
        def _(j):
            pltpu.sync_copy(x_hbm.at[src_v.at[j]], rows_v)
            pltpu.sync_copy(rows_v, acc_sh.at[dst_v.at[j]], add=True)
            pltpu.sync_copy(ones_v, deg_sh.at[dst_v.at[j]], add=True)

        plsc.subcore_barrier()
        o0 = s * SHARE
        pltpu.sync_copy(acc_sh.at[pl.ds(o0, SHARE)],
                        acc_out.at[r, c, pl.ds(o0, SHARE)])
        pltpu.sync_copy(deg_sh.at[pl.ds(o0, SHARE)],
                        deg_out.at[r, c, pl.ds(o0, SHARE)])
        plsc.subcore_barrier()


_BN = 1280


def _tc_body(acc_ref, deg_ref, w_ref, o_ref):
    h = jnp.zeros((_BN, D_FEAT), jnp.float32)
    for r in range(N_REL):
        a = acc_ref[r, 0] + acc_ref[r, 1]
        d = jnp.maximum(deg_ref[r, 0] + deg_ref[r, 1], 1.0)
        h = h + jnp.dot(a / d, w_ref[r],
                        preferred_element_type=jnp.float32,
                        precision=lax.Precision.HIGHEST)
    o_ref[...] = h


def _tc_combine(acc, deg, w):
    return pl.pallas_call(
        _tc_body,
        grid=(ACC_ROWS // _BN,),
        in_specs=[
            pl.BlockSpec((N_REL, NC, _BN, D_FEAT), lambda i: (0, 0, i, 0)),
            pl.BlockSpec((N_REL, NC, _BN, 1), lambda i: (0, 0, i, 0)),
            pl.BlockSpec((N_REL, D_FEAT, D_FEAT), lambda i: (0, 0, 0)),
        ],
        out_specs=pl.BlockSpec((_BN, D_FEAT), lambda i: (i, 0)),
        out_shape=jax.ShapeDtypeStruct((ACC_ROWS, D_FEAT), jnp.float32),
    )(acc, deg, w)


def kernel(x, edge_index_rel0, edge_index_rel1, edge_index_rel2,
           W_rel0, W_rel1, W_rel2):
    src = jnp.stack([edge_index_rel0[0], edge_index_rel1[0],
                     edge_index_rel2[0]]).astype(jnp.int32)
    dst = jnp.stack([edge_index_rel0[1], edge_index_rel1[1],
                     edge_index_rel2[1]]).astype(jnp.int32)
    src = jnp.pad(src, ((0, 0), (0, PAD_E)))
    pad_dst = jnp.broadcast_to(
        N_NODES + (jnp.arange(PAD_E, dtype=jnp.int32) % (ACC_ROWS - N_NODES)),
        (N_REL, PAD_E))
    dst = jnp.concatenate([dst, pad_dst], axis=1)
    src = src.reshape(N_REL, ROWS, CHUNK)
    dst = dst.reshape(N_REL, ROWS, CHUNK)

    ones = jnp.ones((CHUNK,), jnp.float32)
    zacc = jnp.zeros((64, D_FEAT), jnp.float32)
    zdeg = jnp.zeros((SHARE,), jnp.float32)

    acc, deg = _sc_aggregate(x, src, dst, ones, zacc, zdeg)
    deg = deg.reshape(N_REL, NC, ACC_ROWS, 1)
    w = jnp.stack([W_rel0, W_rel1, W_rel2])
    return _tc_combine(acc, deg, w)[:N_NODES]


# speedup vs baseline: 3.4865x; 1.1452x over previous
import functools

import jax
import jax.numpy as jnp
from jax import lax
from jax.experimental import pallas as pl
from jax.experimental.pallas import tpu as pltpu
from jax.experimental.pallas import tpu_sc as plsc

N_NODES = 10000
D_FEAT = 128
N_EDGES = 160000
N_REL = 3

NC, NS = 2, 16
CHUNK = 128
ROWS = 1280
ROWS_PER_TILE = ROWS // (NC * NS)
PAD_E = ROWS * CHUNK - N_EDGES
ACC_ROWS = 10240
SHARE = ACC_ROWS // NS

_mesh = plsc.VectorSubcoreMesh(core_axis_name="c", subcore_axis_name="s")


@functools.partial(
    pl.kernel,
    out_type=(
        jax.ShapeDtypeStruct((N_REL, NC, ACC_ROWS, D_FEAT), jnp.float32),
        jax.ShapeDtypeStruct((N_REL, NC, ACC_ROWS), jnp.float32),
    ),
    mesh=_mesh,
    scratch_types=[
        pltpu.VMEM((ROWS_PER_TILE, CHUNK), jnp.int32),
        pltpu.VMEM((ROWS_PER_TILE, CHUNK), jnp.int32),
        pltpu.VMEM((CHUNK, D_FEAT), jnp.float32),
        pltpu.VMEM((CHUNK, D_FEAT), jnp.float32),
        pltpu.VMEM((CHUNK,), jnp.float32),
        pltpu.VMEM((32, D_FEAT), jnp.float32),
        pltpu.VMEM((SHARE,), jnp.float32),
        pltpu.VMEM_SHARED((ACC_ROWS, D_FEAT), jnp.float32),
        pltpu.VMEM_SHARED((ACC_ROWS,), jnp.float32),
        pltpu.SemaphoreType.DMA,
        pltpu.SemaphoreType.DMA,
        pltpu.SemaphoreType.DMA,
        pltpu.SemaphoreType.DMA,
        pltpu.SemaphoreType.DMA,
    ],
)
def _sc_aggregate(x_hbm, src_hbm, dst_hbm, ones_hbm, zacc_hbm, zdeg_hbm,
                  acc_out, deg_out,
                  src_v, dst_v, rows_v0, rows_v1, ones_v, zacc_v, zdeg_v,
                  acc_sh, deg_sh, sem_g0, sem_g1, sem_s0, sem_s1, sem_d):
    c = lax.axis_index("c")
    s = lax.axis_index("s")
    wid = s * NC + c

    pltpu.sync_copy(ones_hbm, ones_v)
    pltpu.sync_copy(zacc_hbm, zacc_v)
    pltpu.sync_copy(zdeg_hbm, zdeg_v)

    for r in range(N_REL):
        z0 = s * SHARE

        @pl.loop(0, SHARE // 32)
        def _(k):
            pltpu.sync_copy(zacc_v, acc_sh.at[pl.ds(z0 + k * 32, 32)])

        pltpu.sync_copy(zdeg_v, deg_sh.at[pl.ds(z0, SHARE)])
        row0 = wid * ROWS_PER_TILE
        pltpu.sync_copy(src_hbm.at[r, pl.ds(row0, ROWS_PER_TILE)], src_v)
        pltpu.sync_copy(dst_hbm.at[r, pl.ds(row0, ROWS_PER_TILE)], dst_v)
        plsc.subcore_barrier()

        bufs = (rows_v0, rows_v1)
        sem_g = (sem_g0, sem_g1)
        sem_s = (sem_s0, sem_s1)
        for b in range(2):
            pltpu.async_copy(x_hbm.at[src_v.at[b]], bufs[b], sem_g[b])

        @pl.loop(0, ROWS_PER_TILE // 2)
        def _(t):
            for b in range(2):
                j = t * 2 + b
                pltpu.make_async_copy(
                    x_hbm.at[src_v.at[j]], bufs[b], sem_g[b]).wait()
                pltpu.async_copy(ones_v, deg_sh.at[dst_v.at[j]], sem_d,
                                 add=True)
                pltpu.async_copy(bufs[b], acc_sh.at[dst_v.at[j]], sem_s[b],
                                 add=True).wait()

                @pl.when(j < ROWS_PER_TILE - 2)
                def _():
                    pltpu.async_copy(
                        x_hbm.at[src_v.at[j + 2]], bufs[b], sem_g[b])

        @pl.loop(0, ROWS_PER_TILE)
        def _(j):
            pltpu.make_async_copy(
                ones_v, deg_sh.at[dst_v.at[0]], sem_d).wait()

        plsc.subcore_barrier()
        o0 = s * SHARE
        pltpu.sync_copy(acc_sh.at[pl.ds(o0, SHARE)],
                        acc_out.at[r, c, pl.ds(o0, SHARE)])
        pltpu.sync_copy(deg_sh.at[pl.ds(o0, SHARE)],
                        deg_out.at[r, c, pl.ds(o0, SHARE)])
        plsc.subcore_barrier()


_BN = 1280


def _tc_body(acc_ref, deg_ref, w_ref, o_ref):
    h = jnp.zeros((_BN, D_FEAT), jnp.float32)
    for r in range(N_REL):
        a = acc_ref[r, 0] + acc_ref[r, 1]
        d = jnp.maximum(deg_ref[r, 0] + deg_ref[r, 1], 1.0)
        h = h + jnp.dot(a / d, w_ref[r],
                        preferred_element_type=jnp.float32,
                        precision=lax.Precision.HIGHEST)
    o_ref[...] = h


def _tc_combine(acc, deg, w):
    return pl.pallas_call(
        _tc_body,
        grid=(ACC_ROWS // _BN,),
        in_specs=[
            pl.BlockSpec((N_REL, NC, _BN, D_FEAT), lambda i: (0, 0, i, 0)),
            pl.BlockSpec((N_REL, NC, _BN, 1), lambda i: (0, 0, i, 0)),
            pl.BlockSpec((N_REL, D_FEAT, D_FEAT), lambda i: (0, 0, 0)),
        ],
        out_specs=pl.BlockSpec((_BN, D_FEAT), lambda i: (i, 0)),
        out_shape=jax.ShapeDtypeStruct((ACC_ROWS, D_FEAT), jnp.float32),
    )(acc, deg, w)


def kernel(x, edge_index_rel0, edge_index_rel1, edge_index_rel2,
           W_rel0, W_rel1, W_rel2):
    src = jnp.stack([edge_index_rel0[0], edge_index_rel1[0],
                     edge_index_rel2[0]]).astype(jnp.int32)
    dst = jnp.stack([edge_index_rel0[1], edge_index_rel1[1],
                     edge_index_rel2[1]]).astype(jnp.int32)
    src = jnp.pad(src, ((0, 0), (0, PAD_E)))
    pad_dst = jnp.broadcast_to(
        N_NODES + (jnp.arange(PAD_E, dtype=jnp.int32) % (ACC_ROWS - N_NODES)),
        (N_REL, PAD_E))
    dst = jnp.concatenate([dst, pad_dst], axis=1)
    src = src.reshape(N_REL, ROWS, CHUNK)
    dst = dst.reshape(N_REL, ROWS, CHUNK)

    ones = jnp.ones((CHUNK,), jnp.float32)
    zacc = jnp.zeros((32, D_FEAT), jnp.float32)
    zdeg = jnp.zeros((SHARE,), jnp.float32)

    acc, deg = _sc_aggregate(x, src, dst, ones, zacc, zdeg)
    deg = deg.reshape(N_REL, NC, ACC_ROWS, 1)
    w = jnp.stack([W_rel0, W_rel1, W_rel2])
    return _tc_combine(acc, deg, w)[:N_NODES]


# speedup vs baseline: 3.6575x; 1.0490x over previous
import functools

import jax
import jax.numpy as jnp
from jax import lax
from jax.experimental import pallas as pl
from jax.experimental.pallas import tpu as pltpu
from jax.experimental.pallas import tpu_sc as plsc

N_NODES = 10000
D_FEAT = 128
N_EDGES = 160000
N_REL = 3

NC, NS = 2, 16
CHUNK = 128
ROWS = 1280
RPT0 = 64
RPT1 = 16
SUB = 32
PAD_E = ROWS * CHUNK - N_EDGES
ACC_ROWS = 10240
SHARE = ACC_ROWS // NS

_mesh = plsc.VectorSubcoreMesh(core_axis_name="c", subcore_axis_name="s")


@functools.partial(
    pl.kernel,
    out_type=(
        jax.ShapeDtypeStruct((N_REL, NC, ACC_ROWS, D_FEAT), jnp.float32),
        jax.ShapeDtypeStruct((N_REL, NC, ACC_ROWS), jnp.float32),
    ),
    mesh=_mesh,
    scratch_types=[
        pltpu.VMEM((SUB, CHUNK), jnp.int32),
        pltpu.VMEM((SUB, CHUNK), jnp.int32),
        pltpu.VMEM((CHUNK, D_FEAT), jnp.float32),
        pltpu.VMEM((CHUNK, D_FEAT), jnp.float32),
        pltpu.VMEM((CHUNK,), jnp.float32),
        pltpu.VMEM((16, D_FEAT), jnp.float32),
        pltpu.VMEM((SHARE,), jnp.float32),
        pltpu.VMEM_SHARED((ACC_ROWS, D_FEAT), jnp.float32),
        pltpu.VMEM_SHARED((ACC_ROWS,), jnp.float32),
        pltpu.SemaphoreType.DMA,
        pltpu.SemaphoreType.DMA,
        pltpu.SemaphoreType.DMA,
        pltpu.SemaphoreType.DMA,
        pltpu.SemaphoreType.DMA,
    ],
)
def _sc_aggregate(x_hbm, src_hbm, dst_hbm, ones_hbm, zacc_hbm, zdeg_hbm,
                  acc_out, deg_out,
                  src_v, dst_v, rows_v0, rows_v1, ones_v, zacc_v, zdeg_v,
                  acc_sh, deg_sh, sem_g0, sem_g1, sem_s0, sem_s1, sem_d):
    c = lax.axis_index("c")
    s = lax.axis_index("s")
    wid = s * NC + c

    pltpu.sync_copy(ones_hbm, ones_v)
    pltpu.sync_copy(zacc_hbm, zacc_v)
    pltpu.sync_copy(zdeg_hbm, zdeg_v)

    for r in range(N_REL):
        z0 = s * SHARE

        @pl.loop(0, SHARE // 16)
        def _(k):
            pltpu.sync_copy(zacc_v, acc_sh.at[pl.ds(z0 + k * 16, 16)])

        pltpu.sync_copy(zdeg_v, deg_sh.at[pl.ds(z0, SHARE)])
        plsc.subcore_barrier()

        bufs = (rows_v0, rows_v1)
        sem_g = (sem_g0, sem_g1)
        sem_s = (sem_s0, sem_s1)

        def run_block(row_base, n):
            pltpu.sync_copy(src_hbm.at[r, pl.ds(row_base, n)],
                            src_v.at[pl.ds(0, n)])
            pltpu.sync_copy(dst_hbm.at[r, pl.ds(row_base, n)],
                            dst_v.at[pl.ds(0, n)])
            for b in range(2):
                pltpu.async_copy(x_hbm.at[src_v.at[b]], bufs[b], sem_g[b])

            @pl.loop(0, n // 2)
            def _(t):
                for b in range(2):
                    j = t * 2 + b
                    pltpu.make_async_copy(
                        x_hbm.at[src_v.at[j]], bufs[b], sem_g[b]).wait()
                    pltpu.async_copy(ones_v, deg_sh.at[dst_v.at[j]], sem_d,
                                     add=True)
                    pltpu.async_copy(bufs[b], acc_sh.at[dst_v.at[j]],
                                     sem_s[b], add=True).wait()

                    @pl.when(j < n - 2)
                    def _():
                        pltpu.async_copy(
                            x_hbm.at[src_v.at[j + 2]], bufs[b], sem_g[b])

            @pl.loop(0, n)
            def _(j):
                pltpu.make_async_copy(
                    ones_v, deg_sh.at[dst_v.at[0]], sem_d).wait()

        @pl.when(c == 0)
        def _():
            run_block(s * RPT0, SUB)
            run_block(s * RPT0 + SUB, SUB)

        @pl.when(c == 1)
        def _():
            run_block(NS * RPT0 + s * RPT1, RPT1)

        plsc.subcore_barrier()
        o0 = s * SHARE
        pltpu.sync_copy(acc_sh.at[pl.ds(o0, SHARE)],
                        acc_out.at[r, c, pl.ds(o0, SHARE)])
        pltpu.sync_copy(deg_sh.at[pl.ds(o0, SHARE)],
                        deg_out.at[r, c, pl.ds(o0, SHARE)])
        plsc.subcore_barrier()


_BN = 1280


def _tc_body(acc_ref, deg_ref, w_ref, o_ref):
    h = jnp.zeros((_BN, D_FEAT), jnp.float32)
    for r in range(N_REL):
        a = acc_ref[r, 0] + acc_ref[r, 1]
        d = jnp.maximum(deg_ref[r, 0] + deg_ref[r, 1], 1.0)
        h = h + jnp.dot(a / d, w_ref[r],
                        preferred_element_type=jnp.float32,
                        precision=lax.Precision.HIGHEST)
    o_ref[...] = h


def _tc_combine(acc, deg, w):
    return pl.pallas_call(
        _tc_body,
        grid=(ACC_ROWS // _BN,),
        in_specs=[
            pl.BlockSpec((N_REL, NC, _BN, D_FEAT), lambda i: (0, 0, i, 0)),
            pl.BlockSpec((N_REL, NC, _BN, 1), lambda i: (0, 0, i, 0)),
            pl.BlockSpec((N_REL, D_FEAT, D_FEAT), lambda i: (0, 0, 0)),
        ],
        out_specs=pl.BlockSpec((_BN, D_FEAT), lambda i: (i, 0)),
        out_shape=jax.ShapeDtypeStruct((ACC_ROWS, D_FEAT), jnp.float32),
    )(acc, deg, w)


def kernel(x, edge_index_rel0, edge_index_rel1, edge_index_rel2,
           W_rel0, W_rel1, W_rel2):
    src = jnp.stack([edge_index_rel0[0], edge_index_rel1[0],
                     edge_index_rel2[0]]).astype(jnp.int32)
    dst = jnp.stack([edge_index_rel0[1], edge_index_rel1[1],
                     edge_index_rel2[1]]).astype(jnp.int32)
    src = jnp.pad(src, ((0, 0), (0, PAD_E)))
    pad_dst = jnp.broadcast_to(
        N_NODES + (jnp.arange(PAD_E, dtype=jnp.int32) % (ACC_ROWS - N_NODES)),
        (N_REL, PAD_E))
    dst = jnp.concatenate([dst, pad_dst], axis=1)
    src = src.reshape(N_REL, ROWS, CHUNK)
    dst = dst.reshape(N_REL, ROWS, CHUNK)

    ones = jnp.ones((CHUNK,), jnp.float32)
    zacc = jnp.zeros((16, D_FEAT), jnp.float32)
    zdeg = jnp.zeros((SHARE,), jnp.float32)

    acc, deg = _sc_aggregate(x, src, dst, ones, zacc, zdeg)
    deg = deg.reshape(N_REL, NC, ACC_ROWS, 1)
    w = jnp.stack([W_rel0, W_rel1, W_rel2])
    return _tc_combine(acc, deg, w)[:N_NODES]
